# baseline (speedup 1.0000x reference)
import jax
import jax.numpy as jnp
from jax import lax
from jax.experimental import pallas as pl
from jax.experimental.pallas import tpu as pltpu
from jax.experimental.pallas import tpu_sc as plsc

B = 16
S = 4096
D = 1024
LANES = 16
RSC = 2048
CH = 512
N_CH = (RSC // 2) // CH
STEPS = (CH // 2) // LANES
RB = 256
N_STREAMS = (S - RSC) // RB


def _sc_count_body(sent_hbm, cnt_hbm, chunk_v, acc_v, partner_v,
                   cnt_sh, slab_sh):
    c = lax.axis_index("c")
    s = lax.axis_index("s")
    b = c * 8 + s // 2
    h = s % 2

    lane = lax.iota(jnp.int32, LANES)
    zeros = jnp.zeros((LANES,), jnp.int32)
    ones = jnp.ones((LANES,), jnp.int32)

    acc = jnp.zeros((LANES,), jnp.int32)
    for k in range(N_CH):
        row0 = h * (RSC // 2) + k * CH
        pltpu.sync_copy(sent_hbm.at[b, pl.ds(row0, CH), pl.ds(0, 128)],
                        chunk_v)

        def step(i, a):
            rows = 2 * (i * LANES + lane) + 1
            vals = plsc.load_gather(chunk_v, [rows, zeros])
            return a + jnp.where(vals != 0.0, ones, zeros)

        acc = lax.fori_loop(0, STEPS, step, acc)

    acc_v[...] = acc
    pltpu.sync_copy(acc_v, cnt_sh.at[s])
    plsc.subcore_barrier()
    pltpu.sync_copy(cnt_sh.at[s ^ 1], partner_v)
    count_sc = jnp.sum(acc_v[...] + partner_v[...])

    acc_v[...] = jnp.where(lane == 0, count_sc, 0)

    @pl.when(h == 0)
    def _():
        pltpu.sync_copy(acc_v, slab_sh.at[s // 2, 0, pl.ds(0, LANES)])

    plsc.subcore_barrier()
    out0 = pl.multiple_of(c * 8, 8)

    @pl.when(s == 0)
    def _():
        pltpu.sync_copy(slab_sh, cnt_hbm.at[pl.ds(out0, 8)])


def _tc_count_body(*refs):
    (*x_refs, cnt_ref) = refs
    rows = lax.broadcasted_iota(jnp.int32, (RB, 128), 0)
    odd = rows % 2 == 1
    nz = jnp.zeros((RB, 128), jnp.int32)
    for x_ref in x_refs:
        nz += jnp.where((x_ref[0] != 0.0) & odd, 1, 0)
    cnt_ref[0, 0] = jnp.sum(nz, axis=0)


def _gather_map0(b, sc_ref, tc_ref):
    return (b, _row_of_total(sc_ref, tc_ref, b, 0) // 8, 0)


def _gather_map1(b, sc_ref, tc_ref):
    return (b, _row_of_total(sc_ref, tc_ref, b, 1) // 8, 0)


def _row_of_total(sc_ref, tc_ref, b, h):
    r = sc_ref[b, 0] + tc_ref[b, 0] - 2 + h
    return jnp.where(r < 0, r + S, r)


def _tc_gather_body(sc_ref, tc_ref, x0_ref, x1_ref, o0_ref, o1_ref):
    b = pl.program_id(0)
    q0 = _row_of_total(sc_ref, tc_ref, b, 0) % 8
    q1 = _row_of_total(sc_ref, tc_ref, b, 1) % 8
    for j in range(8):
        @pl.when(q0 == j)
        def _(j=j):
            o0_ref[0, 0] = x0_ref[0, j]

        @pl.when(q1 == j)
        def _(j=j):
            o1_ref[0, 0] = x1_ref[0, j]


@jax.jit
def kernel(sent_emb, entity_emb):
    del entity_emb

    sc_counts = pl.kernel(
        _sc_count_body,
        out_type=jax.ShapeDtypeStruct((B, 8, 128), jnp.int32),
        mesh=plsc.VectorSubcoreMesh(core_axis_name="c", subcore_axis_name="s"),
        scratch_types=[
            pltpu.VMEM((CH, 128), jnp.float32),
            pltpu.VMEM((LANES,), jnp.int32),
            pltpu.VMEM((LANES,), jnp.int32),
            pltpu.VMEM_SHARED((B, LANES), jnp.int32),
            pltpu.VMEM_SHARED((8, 8, 128), jnp.int32),
        ],
        compiler_params=pltpu.CompilerParams(needs_layout_passes=False),
    )(sent_emb)

    tc_counts = pl.pallas_call(
        _tc_count_body,
        grid=(B,),
        in_specs=[
            pl.BlockSpec((1, RB, 128), lambda b, j=j: (b, RSC // RB + j, 0))
            for j in range(N_STREAMS)
        ],
        out_specs=pl.BlockSpec((1, 1, 128), lambda b: (b, 0, 0)),
        out_shape=jax.ShapeDtypeStruct((B, 1, 128), jnp.int32),
    )(*([sent_emb] * N_STREAMS))

    sc_scalar = sc_counts[:, 0, :1]
    tc_scalar = tc_counts[:, 0, :1]

    out3 = jax.ShapeDtypeStruct((B, 1, D), jnp.float32)
    a_hat3, a_n3 = pl.pallas_call(
        _tc_gather_body,
        grid_spec=pltpu.PrefetchScalarGridSpec(
            num_scalar_prefetch=2,
            grid=(B,),
            in_specs=[
                pl.BlockSpec((1, 8, D), _gather_map0),
                pl.BlockSpec((1, 8, D), _gather_map1),
            ],
            out_specs=[
                pl.BlockSpec((1, 1, D), lambda b, sc, tc: (b, 0, 0)),
                pl.BlockSpec((1, 1, D), lambda b, sc, tc: (b, 0, 0)),
            ],
        ),
        out_shape=(out3, out3),
    )(sc_scalar, tc_scalar, sent_emb, sent_emb)

    sent_a_hat_n = a_hat3.reshape(B, D)
    sent_a_n = a_n3.reshape(B, D)
    return sent_a_hat_n, sent_a_n, jnp.asarray(0.0, dtype=jnp.float32)


# speedup vs baseline: 1.2570x; 1.2570x over previous
import jax
import jax.numpy as jnp
from jax import lax
from jax.experimental import pallas as pl
from jax.experimental.pallas import tpu as pltpu
from jax.experimental.pallas import tpu_sc as plsc

B = 16
S = 4096
D = 1024
LANES = 16
RSC = 1792
CH = RSC // 4
N_CH = (RSC // 2) // CH
STEPS = (CH // 2) // LANES
RB = 256
N_STREAMS = (S - RSC) // RB


def _sc_count_body(sent_hbm, cnt_hbm, chunk_v, acc_v, partner_v,
                   cnt_sh, slab_sh):
    c = lax.axis_index("c")
    s = lax.axis_index("s")
    b = c * 8 + s // 2
    h = s % 2

    lane = lax.iota(jnp.int32, LANES)
    zeros = jnp.zeros((LANES,), jnp.int32)
    ones = jnp.ones((LANES,), jnp.int32)

    acc = jnp.zeros((LANES,), jnp.int32)
    for k in range(N_CH):
        row0 = h * (RSC // 2) + k * CH
        pltpu.sync_copy(sent_hbm.at[b, pl.ds(row0, CH), pl.ds(0, 128)],
                        chunk_v)

        def step(i, a):
            rows = 2 * (i * LANES + lane) + 1
            vals = plsc.load_gather(chunk_v, [rows, zeros])
            return a + jnp.where(vals != 0.0, ones, zeros)

        acc = lax.fori_loop(0, STEPS, step, acc)

    acc_v[...] = acc
    pltpu.sync_copy(acc_v, cnt_sh.at[s])
    plsc.subcore_barrier()
    pltpu.sync_copy(cnt_sh.at[s ^ 1], partner_v)
    count_sc = jnp.sum(acc_v[...] + partner_v[...])

    acc_v[...] = jnp.where(lane == 0, count_sc, 0)

    @pl.when(h == 0)
    def _():
        pltpu.sync_copy(acc_v, slab_sh.at[s // 2, 0, pl.ds(0, LANES)])

    plsc.subcore_barrier()
    out0 = pl.multiple_of(c * 8, 8)

    @pl.when(s == 0)
    def _():
        pltpu.sync_copy(slab_sh, cnt_hbm.at[pl.ds(out0, 8)])


def _tc_count_body(*refs):
    (*x_refs, cnt_ref) = refs
    rows = lax.broadcasted_iota(jnp.int32, (RB, 128), 0)
    odd = rows % 2 == 1
    for g in range(2):
        nz = jnp.zeros((RB, 128), jnp.int32)
        for x_ref in x_refs:
            nz += jnp.where((x_ref[g] != 0.0) & odd, 1, 0)
        cnt_ref[g, 0] = jnp.sum(nz, axis=0)


def _row_of_total(sc_ref, tc_ref, b, h):
    r = sc_ref[b, 0, 0] + tc_ref[b, 0, 0] - 2 + h
    return jnp.where(r < 0, r + S, r)


def _mk_gather_map(b, h):
    def gmap(i, sc_ref, tc_ref):
        return (b, _row_of_total(sc_ref, tc_ref, b, h) // 8, 0)
    return gmap


def _tc_gather_body(sc_ref, tc_ref, *refs):
    (*x_refs, o0_ref, o1_ref) = refs
    for b in range(B):
        q0 = _row_of_total(sc_ref, tc_ref, b, 0) % 8
        q1 = _row_of_total(sc_ref, tc_ref, b, 1) % 8
        for j in range(8):
            @pl.when(q0 == j)
            def _(b=b, j=j):
                o0_ref[b] = x_refs[2 * b][0, j]

            @pl.when(q1 == j)
            def _(b=b, j=j):
                o1_ref[b] = x_refs[2 * b + 1][0, j]


@jax.jit
def kernel(sent_emb, entity_emb):
    del entity_emb

    sc_counts = pl.kernel(
        _sc_count_body,
        out_type=jax.ShapeDtypeStruct((B, 8, 128), jnp.int32),
        mesh=plsc.VectorSubcoreMesh(core_axis_name="c", subcore_axis_name="s"),
        scratch_types=[
            pltpu.VMEM((CH, 128), jnp.float32),
            pltpu.VMEM((LANES,), jnp.int32),
            pltpu.VMEM((LANES,), jnp.int32),
            pltpu.VMEM_SHARED((B, LANES), jnp.int32),
            pltpu.VMEM_SHARED((8, 8, 128), jnp.int32),
        ],
        compiler_params=pltpu.CompilerParams(needs_layout_passes=False),
    )(sent_emb)

    tc_counts = pl.pallas_call(
        _tc_count_body,
        grid=(B // 2,),
        in_specs=[
            pl.BlockSpec((2, RB, 128), lambda g, j=j: (g, RSC // RB + j, 0))
            for j in range(N_STREAMS)
        ],
        out_specs=pl.BlockSpec((2, 1, 128), lambda g: (g, 0, 0)),
        out_shape=jax.ShapeDtypeStruct((B, 1, 128), jnp.int32),
    )(*([sent_emb] * N_STREAMS))

    out2 = jax.ShapeDtypeStruct((B, D), jnp.float32)
    sent_a_hat_n, sent_a_n = pl.pallas_call(
        _tc_gather_body,
        grid_spec=pltpu.PrefetchScalarGridSpec(
            num_scalar_prefetch=2,
            grid=(1,),
            in_specs=[
                pl.BlockSpec((1, 8, D), _mk_gather_map(b, h))
                for b in range(B) for h in range(2)
            ],
            out_specs=[
                pl.BlockSpec((B, D), lambda i, sc, tc: (0, 0)),
                pl.BlockSpec((B, D), lambda i, sc, tc: (0, 0)),
            ],
        ),
        out_shape=(out2, out2),
    )(sc_counts, tc_counts, *([sent_emb] * (2 * B)))

    return sent_a_hat_n, sent_a_n, jnp.asarray(0.0, dtype=jnp.float32)


# speedup vs baseline: 1.2908x; 1.0269x over previous
import jax
import jax.numpy as jnp
from jax import lax
from jax.experimental import pallas as pl
from jax.experimental.pallas import tpu as pltpu
from jax.experimental.pallas import tpu_sc as plsc

B = 16
S = 4096
D = 1024
LANES = 16
RSC = 1792
CH = RSC // 4
N_CH = (RSC // 2) // CH
STEPS = (CH // 2) // LANES
RB = 256
N_STREAMS = (S - RSC) // RB


def _sc_count_body(sent_hbm, cnt_hbm, chunk_v, acc_v, partner_v,
                   cnt_sh, slab_sh):
    c = lax.axis_index("c")
    s = lax.axis_index("s")
    b = c * 8 + s // 2
    h = s % 2

    lane = lax.iota(jnp.int32, LANES)
    zeros = jnp.zeros((LANES,), jnp.int32)
    ones = jnp.ones((LANES,), jnp.int32)

    acc = jnp.zeros((LANES,), jnp.int32)
    for k in range(N_CH):
        row0 = h * (RSC // 2) + k * CH
        pltpu.sync_copy(sent_hbm.at[b, pl.ds(row0, CH), pl.ds(0, 128)],
                        chunk_v)

        def step(i, a):
            rows = 2 * (i * LANES + lane) + 1
            vals = plsc.load_gather(chunk_v, [rows, zeros])
            return a + jnp.where(vals != 0.0, ones, zeros)

        acc = lax.fori_loop(0, STEPS, step, acc)

    acc_v[...] = acc
    pltpu.sync_copy(acc_v, cnt_sh.at[s])
    plsc.subcore_barrier()
    pltpu.sync_copy(cnt_sh.at[s ^ 1], partner_v)
    count_sc = jnp.sum(acc_v[...] + partner_v[...])

    acc_v[...] = jnp.where(lane == 0, count_sc, 0)

    @pl.when(h == 0)
    def _():
        pltpu.sync_copy(acc_v, slab_sh.at[s // 2, 0, pl.ds(0, LANES)])

    plsc.subcore_barrier()
    out0 = pl.multiple_of(c * 8, 8)

    @pl.when(s == 0)
    def _():
        pltpu.sync_copy(slab_sh, cnt_hbm.at[pl.ds(out0, 8)])


def _tc_count_body(*refs):
    (*x_refs, cnt_ref) = refs
    rows = lax.broadcasted_iota(jnp.int32, (RB, 128), 0)
    odd = rows % 2 == 1
    for g in range(2):
        nz = jnp.zeros((RB, 128), jnp.int32)
        for x_ref in x_refs:
            nz += jnp.where((x_ref[g] != 0.0) & odd, 1, 0)
        cnt_ref[g, 0] = jnp.sum(nz, axis=0)


def _row_of_total(sc_ref, tc_ref, b, h):
    r = sc_ref[b, 0, 0] + tc_ref[b, 0, 0] - 2 + h
    return jnp.where(r < 0, r + S, r)


def _mk_gather_map(b, h):
    def gmap(i, sc_ref, tc_ref):
        return (b, _row_of_total(sc_ref, tc_ref, b, h) // 8, 0)
    return gmap


def _tc_gather_body(sc_ref, tc_ref, *refs):
    (*x_refs, o0_ref, o1_ref) = refs
    sub = lax.broadcasted_iota(jnp.int32, (8, D), 0)
    for b in range(B):
        for h, o_ref in ((0, o0_ref), (1, o1_ref)):
            q = _row_of_total(sc_ref, tc_ref, b, h) % 8
            x = x_refs[2 * b + h][0]
            o_ref[b] = jnp.sum(jnp.where(sub == q, x, 0.0), axis=0)


@jax.jit
def kernel(sent_emb, entity_emb):
    del entity_emb

    sc_counts = pl.kernel(
        _sc_count_body,
        out_type=jax.ShapeDtypeStruct((B, 8, 128), jnp.int32),
        mesh=plsc.VectorSubcoreMesh(core_axis_name="c", subcore_axis_name="s"),
        scratch_types=[
            pltpu.VMEM((CH, 128), jnp.float32),
            pltpu.VMEM((LANES,), jnp.int32),
            pltpu.VMEM((LANES,), jnp.int32),
            pltpu.VMEM_SHARED((B, LANES), jnp.int32),
            pltpu.VMEM_SHARED((8, 8, 128), jnp.int32),
        ],
        compiler_params=pltpu.CompilerParams(needs_layout_passes=False),
    )(sent_emb)

    tc_counts = pl.pallas_call(
        _tc_count_body,
        grid=(B // 2,),
        in_specs=[
            pl.BlockSpec((2, RB, 128), lambda g, j=j: (g, RSC // RB + j, 0))
            for j in range(N_STREAMS)
        ],
        out_specs=pl.BlockSpec((2, 1, 128), lambda g: (g, 0, 0)),
        out_shape=jax.ShapeDtypeStruct((B, 1, 128), jnp.int32),
    )(*([sent_emb] * N_STREAMS))

    out2 = jax.ShapeDtypeStruct((B, D), jnp.float32)
    sent_a_hat_n, sent_a_n = pl.pallas_call(
        _tc_gather_body,
        grid_spec=pltpu.PrefetchScalarGridSpec(
            num_scalar_prefetch=2,
            grid=(1,),
            in_specs=[
                pl.BlockSpec((1, 8, D), _mk_gather_map(b, h))
                for b in range(B) for h in range(2)
            ],
            out_specs=[
                pl.BlockSpec((B, D), lambda i, sc, tc: (0, 0)),
                pl.BlockSpec((B, D), lambda i, sc, tc: (0, 0)),
            ],
        ),
        out_shape=(out2, out2),
    )(sc_counts, tc_counts, *([sent_emb] * (2 * B)))

    return sent_a_hat_n, sent_a_n, jnp.asarray(0.0, dtype=jnp.float32)


# speedup vs baseline: 1.3058x; 1.0116x over previous
import jax
import jax.numpy as jnp
from jax import lax
from jax.experimental import pallas as pl
from jax.experimental.pallas import tpu as pltpu
from jax.experimental.pallas import tpu_sc as plsc

B = 16
S = 4096
D = 1024
LANES = 16
RSC = 1792
CH = RSC // 2
N_CH = (RSC // 2) // CH
STEPS = (CH // 2) // LANES
RB = 256
N_STREAMS = (S - RSC) // RB


def _sc_count_body(sent_hbm, cnt_hbm, chunk_v, acc_v, partner_v,
                   cnt_sh, slab_sh):
    c = lax.axis_index("c")
    s = lax.axis_index("s")
    b = c * 8 + s // 2
    h = s % 2

    lane = lax.iota(jnp.int32, LANES)
    zeros = jnp.zeros((LANES,), jnp.int32)
    ones = jnp.ones((LANES,), jnp.int32)

    acc = jnp.zeros((LANES,), jnp.int32)
    for k in range(N_CH):
        row0 = h * (RSC // 2) + k * CH
        pltpu.sync_copy(sent_hbm.at[b, pl.ds(row0, CH), pl.ds(0, 128)],
                        chunk_v)

        def step(i, a):
            rows = 2 * (i * LANES + lane) + 1
            vals = plsc.load_gather(chunk_v, [rows, zeros])
            return a + jnp.where(vals != 0.0, ones, zeros)

        acc = lax.fori_loop(0, STEPS, step, acc)

    acc_v[...] = acc
    pltpu.sync_copy(acc_v, cnt_sh.at[s])
    plsc.subcore_barrier()
    pltpu.sync_copy(cnt_sh.at[s ^ 1], partner_v)
    count_sc = jnp.sum(acc_v[...] + partner_v[...])

    acc_v[...] = jnp.where(lane == 0, count_sc, 0)

    @pl.when(h == 0)
    def _():
        pltpu.sync_copy(acc_v, slab_sh.at[s // 2, 0, pl.ds(0, LANES)])

    plsc.subcore_barrier()
    out0 = pl.multiple_of(c * 8, 8)

    @pl.when(s == 0)
    def _():
        pltpu.sync_copy(slab_sh, cnt_hbm.at[pl.ds(out0, 8)])


def _tc_count_body(*refs):
    (*x_refs, cnt_ref) = refs
    rows = lax.broadcasted_iota(jnp.int32, (RB, 128), 0)
    odd = rows % 2 == 1
    for g in range(2):
        nz = jnp.zeros((RB, 128), jnp.int32)
        for x_ref in x_refs:
            nz += jnp.where((x_ref[g] != 0.0) & odd, 1, 0)
        cnt_ref[g, 0] = jnp.sum(nz, axis=0)


def _row_of_total(sc_ref, tc_ref, b, h):
    r = sc_ref[b, 0, 0] + tc_ref[b, 0, 0] - 2 + h
    return jnp.where(r < 0, r + S, r)


def _mk_gather_map(b, h):
    def gmap(i, sc_ref, tc_ref):
        return (b, _row_of_total(sc_ref, tc_ref, b, h) // 8, 0)
    return gmap


def _tc_gather_body(sc_ref, tc_ref, *refs):
    (*x_refs, o0_ref, o1_ref) = refs
    sub = lax.broadcasted_iota(jnp.int32, (8, D), 0)
    for b in range(B):
        for h, o_ref in ((0, o0_ref), (1, o1_ref)):
            q = _row_of_total(sc_ref, tc_ref, b, h) % 8
            x = x_refs[2 * b + h][0]
            o_ref[b] = jnp.sum(jnp.where(sub == q, x, 0.0), axis=0)


@jax.jit
def kernel(sent_emb, entity_emb):
    del entity_emb

    sc_counts = pl.kernel(
        _sc_count_body,
        out_type=jax.ShapeDtypeStruct((B, 8, 128), jnp.int32),
        mesh=plsc.VectorSubcoreMesh(core_axis_name="c", subcore_axis_name="s"),
        scratch_types=[
            pltpu.VMEM((CH, 128), jnp.float32),
            pltpu.VMEM((LANES,), jnp.int32),
            pltpu.VMEM((LANES,), jnp.int32),
            pltpu.VMEM_SHARED((B, LANES), jnp.int32),
            pltpu.VMEM_SHARED((8, 8, 128), jnp.int32),
        ],
        compiler_params=pltpu.CompilerParams(needs_layout_passes=False),
    )(sent_emb)

    tc_counts = pl.pallas_call(
        _tc_count_body,
        grid=(B // 2,),
        in_specs=[
            pl.BlockSpec((2, RB, 128), lambda g, j=j: (g, RSC // RB + j, 0))
            for j in range(N_STREAMS)
        ],
        out_specs=pl.BlockSpec((2, 1, 128), lambda g: (g, 0, 0)),
        out_shape=jax.ShapeDtypeStruct((B, 1, 128), jnp.int32),
    )(*([sent_emb] * N_STREAMS))

    out2 = jax.ShapeDtypeStruct((B, D), jnp.float32)
    sent_a_hat_n, sent_a_n = pl.pallas_call(
        _tc_gather_body,
        grid_spec=pltpu.PrefetchScalarGridSpec(
            num_scalar_prefetch=2,
            grid=(1,),
            in_specs=[
                pl.BlockSpec((1, 8, D), _mk_gather_map(b, h))
                for b in range(B) for h in range(2)
            ],
            out_specs=[
                pl.BlockSpec((B, D), lambda i, sc, tc: (0, 0)),
                pl.BlockSpec((B, D), lambda i, sc, tc: (0, 0)),
            ],
        ),
        out_shape=(out2, out2),
    )(sc_counts, tc_counts, *([sent_emb] * (2 * B)))

    return sent_a_hat_n, sent_a_n, jnp.asarray(0.0, dtype=jnp.float32)
